# baseline (speedup 1.0000x reference)
import functools

import jax
import jax.numpy as jnp
from jax import lax
from jax.experimental import pallas as pl
from jax.experimental.pallas import tpu as pltpu
from jax.experimental.pallas import tpu_sc as plsc

TOKENS = 16384
N_EMBD = 2048
N_EXPERTS = 8
TOPK = 2
BLOCK = 2048

NUM_CORES = 2
NUM_SUBCORES = 16
LANES = 16
NUM_WORKERS = NUM_CORES * NUM_SUBCORES
TOK_PER_WORKER = TOKENS // NUM_WORKERS
CHUNK = 128


def _scores_block(x_ref, w_ref, scores_ref):
    scores = jnp.dot(x_ref[...], w_ref[...], preferred_element_type=jnp.float32)
    scores_ref[...] = jax.nn.sigmoid(scores)


def _tc_scores(x, weight):
    n_tokens = x.shape[0]
    return pl.pallas_call(
        _scores_block,
        grid=(n_tokens // BLOCK,),
        in_specs=[
            pl.BlockSpec((BLOCK, N_EMBD), lambda i: (i, 0)),
            pl.BlockSpec((N_EMBD, N_EXPERTS), lambda i: (0, 0)),
        ],
        out_specs=pl.BlockSpec((BLOCK, N_EXPERTS), lambda i: (i, 0)),
        out_shape=jax.ShapeDtypeStruct((n_tokens, N_EXPERTS), jnp.float32),
        compiler_params=pltpu.CompilerParams(
            dimension_semantics=("arbitrary",),
        ),
    )(x, weight)


@functools.partial(
    pl.kernel,
    mesh=plsc.VectorSubcoreMesh(core_axis_name="c", subcore_axis_name="s"),
    out_type=[
        jax.ShapeDtypeStruct((TOKENS, TOPK), jnp.float32),
        jax.ShapeDtypeStruct((TOKENS, TOPK), jnp.int32),
    ],
    scratch_types=[
        pltpu.VMEM((CHUNK, N_EXPERTS), jnp.float32),
        pltpu.VMEM((CHUNK, TOPK), jnp.float32),
        pltpu.VMEM((CHUNK, TOPK), jnp.int32),
    ],
    compiler_params=pltpu.CompilerParams(needs_layout_passes=False),
)
def _sc_route(scores_hbm, w_hbm, i_hbm, s_v, w_v, i_v):
    wid = lax.axis_index("s") * NUM_CORES + lax.axis_index("c")
    base = wid * TOK_PER_WORKER

    lane = lax.iota(jnp.int32, LANES)
    zero = jnp.zeros((LANES,), jnp.int32)
    one = jnp.ones((LANES,), jnp.int32)

    def do_chunk(k, carry):
        tok = base + k * CHUNK
        pltpu.sync_copy(scores_hbm.at[pl.ds(tok, CHUNK)], s_v)

        def sub(c, carry2):
            row = c * LANES + lane
            cols = [
                plsc.load_gather(s_v, [row, jnp.full((LANES,), e, jnp.int32)])
                for e in range(N_EXPERTS)
            ]
            m1 = cols[0]
            for e in range(1, N_EXPERTS):
                m1 = jnp.maximum(m1, cols[e])
            i1 = jnp.full((LANES,), N_EXPERTS - 1, jnp.int32)
            for e in range(N_EXPERTS - 2, -1, -1):
                i1 = jnp.where(cols[e] == m1, e, i1)
            rest = [jnp.where(i1 == e, -1.0, cols[e]) for e in range(N_EXPERTS)]
            m2 = rest[0]
            for e in range(1, N_EXPERTS):
                m2 = jnp.maximum(m2, rest[e])
            i2 = jnp.full((LANES,), N_EXPERTS - 1, jnp.int32)
            for e in range(N_EXPERTS - 2, -1, -1):
                i2 = jnp.where(rest[e] == m2, e, i2)
            denom = m1 + m2 + 1e-6
            plsc.store_scatter(w_v, [row, zero], m1 / denom)
            plsc.store_scatter(w_v, [row, one], m2 / denom)
            plsc.store_scatter(i_v, [row, zero], i1)
            plsc.store_scatter(i_v, [row, one], i2)
            return carry2

        lax.fori_loop(0, CHUNK // LANES, sub, 0)

        pltpu.sync_copy(w_v, w_hbm.at[pl.ds(tok, CHUNK)])
        pltpu.sync_copy(i_v, i_hbm.at[pl.ds(tok, CHUNK)])
        return carry

    lax.fori_loop(0, TOK_PER_WORKER // CHUNK, do_chunk, 0)


def kernel(x, weight):
    scores = _tc_scores(x, weight)
    weights, indices = _sc_route(scores)
    return (scores, weights, indices)


# speedup vs baseline: 1.9501x; 1.9501x over previous
import functools

import jax
import jax.numpy as jnp
from jax import lax
from jax.experimental import pallas as pl
from jax.experimental.pallas import tpu as pltpu

TOKENS = 16384
N_EMBD = 2048
N_EXPERTS = 8
TOPK = 2
BLOCK = 2048


def _gate_block(x_ref, w_ref, st_ref, wt_ref, it_ref):
    scores = jnp.dot(x_ref[...], w_ref[...], preferred_element_type=jnp.float32)
    st = jax.nn.sigmoid(scores.T)
    st_ref[...] = st

    row = lax.broadcasted_iota(jnp.int32, st.shape, 0)
    m1 = jnp.max(st, axis=0, keepdims=True)
    i1 = jnp.min(jnp.where(st == m1, row, N_EXPERTS), axis=0, keepdims=True)
    rest = jnp.where(row == i1, -1.0, st)
    m2 = jnp.max(rest, axis=0, keepdims=True)
    i2 = jnp.min(jnp.where(rest == m2, row, N_EXPERTS), axis=0, keepdims=True)

    denom = m1 + m2 + 1e-6
    wt_ref[...] = jnp.concatenate([m1 / denom, m2 / denom], axis=0)
    it_ref[...] = jnp.concatenate([i1, i2], axis=0)


def kernel(x, weight):
    n_tokens = x.shape[0]
    st, wt, it = pl.pallas_call(
        _gate_block,
        grid=(n_tokens // BLOCK,),
        in_specs=[
            pl.BlockSpec((BLOCK, N_EMBD), lambda i: (i, 0)),
            pl.BlockSpec((N_EMBD, N_EXPERTS), lambda i: (0, 0)),
        ],
        out_specs=[
            pl.BlockSpec((N_EXPERTS, BLOCK), lambda i: (0, i)),
            pl.BlockSpec((TOPK, BLOCK), lambda i: (0, i)),
            pl.BlockSpec((TOPK, BLOCK), lambda i: (0, i)),
        ],
        out_shape=[
            jax.ShapeDtypeStruct((N_EXPERTS, n_tokens), jnp.float32),
            jax.ShapeDtypeStruct((TOPK, n_tokens), jnp.float32),
            jax.ShapeDtypeStruct((TOPK, n_tokens), jnp.int32),
        ],
        compiler_params=pltpu.CompilerParams(
            dimension_semantics=("arbitrary",),
        ),
    )(x, weight)
    return (st.T, wt.T, it.T)
